# Initial kernel scaffold; baseline (speedup 1.0000x reference)
import jax
import jax.numpy as jnp
from jax.experimental import pallas as pl


def kernel(x, Wg, bg, gamma, beta):
    raise NotImplementedError("write your pallas kernel here")


# speedup vs baseline: 78.4538x; 78.4538x over previous
import functools
import math

import jax
import jax.numpy as jnp
from jax.experimental import pallas as pl
from jax.experimental.pallas import tpu as pltpu

B, C, H, W_ = 32, 96, 32, 32
N = H * W_
K = 16
INF = 1e10
BN_SCALE = 1.0 / math.sqrt(1.0 + 1e-5)
INV_SQRT2 = 1.0 / math.sqrt(2.0)


def _graph_kernel(x_ref, wg_ref, bg_ref, gamma_ref, beta_ref, out_ref):
    xb = x_ref[0]
    sq = jnp.sum(xb * xb, axis=1, keepdims=True)
    g = jax.lax.dot_general(xb, xb, (((1,), (1,)), ((), ())),
                            preferred_element_type=jnp.float32)
    d2 = sq + jnp.transpose(sq) - 2.0 * g
    rows = jax.lax.broadcasted_iota(jnp.int32, (N, N), 0)
    cols = jax.lax.broadcasted_iota(jnp.int32, (N, N), 1)
    off_diag = rows != cols
    m = jnp.where(off_diag, d2, INF)
    for _ in range(K):
        cur = jnp.min(m, axis=1, keepdims=True)
        m = jnp.where(m <= cur, INF, m)
    mask = jnp.where((m >= 0.5 * INF) & off_diag, 1.0, 0.0)
    h = jnp.dot(xb, wg_ref[...], preferred_element_type=jnp.float32)
    agg = jnp.dot(mask, h, preferred_element_type=jnp.float32) + h
    y = agg * (1.0 / (K + 1)) + bg_ref[...]
    y = y * (gamma_ref[...] * BN_SCALE) + beta_ref[...]
    y = y * 0.5 * (1.0 + jax.lax.erf(y * INV_SQRT2))
    out_ref[0] = y + xb


@jax.jit
def kernel(x, Wg, bg, gamma, beta):
    x_nodes = jnp.transpose(x, (0, 2, 3, 1)).reshape(B, N, C)
    out = pl.pallas_call(
        _graph_kernel,
        grid=(B,),
        in_specs=[
            pl.BlockSpec((1, N, C), lambda b: (b, 0, 0)),
            pl.BlockSpec((C, C), lambda b: (0, 0)),
            pl.BlockSpec((1, C), lambda b: (0, 0)),
            pl.BlockSpec((1, C), lambda b: (0, 0)),
            pl.BlockSpec((1, C), lambda b: (0, 0)),
        ],
        out_specs=pl.BlockSpec((1, N, C), lambda b: (b, 0, 0)),
        out_shape=jax.ShapeDtypeStruct((B, N, C), jnp.float32),
    )(x_nodes, Wg, bg.reshape(1, C), gamma.reshape(1, C), beta.reshape(1, C))
    return out.reshape(B, H, W_, C).transpose(0, 3, 1, 2)


# speedup vs baseline: 83.5193x; 1.0646x over previous
import functools
import math

import jax
import jax.numpy as jnp
from jax.experimental import pallas as pl
from jax.experimental.pallas import tpu as pltpu

B, C, H, W_ = 32, 96, 32, 32
N = H * W_
K = 16
INF = 1e10
BN_SCALE = 1.0 / math.sqrt(1.0 + 1e-5)
INV_SQRT2 = 1.0 / math.sqrt(2.0)


def _graph_kernel(x_ref, wg_ref, bg_ref, gamma_ref, beta_ref, out_ref):
    xb = x_ref[0]
    sq = jnp.sum(xb * xb, axis=1, keepdims=True)
    g = jax.lax.dot_general(xb, xb, (((1,), (1,)), ((), ())),
                            preferred_element_type=jnp.float32)
    d2 = sq + jnp.transpose(sq) - 2.0 * g
    rows = jax.lax.broadcasted_iota(jnp.int32, (N, N), 0)
    cols = jax.lax.broadcasted_iota(jnp.int32, (N, N), 1)
    off_diag = rows != cols
    m = jnp.where(off_diag, d2, INF)
    cur = jnp.min(m, axis=1, keepdims=True)
    for _ in range(K - 1):
        cur = jnp.min(jnp.where(m > cur, m, INF), axis=1, keepdims=True)
    mask = jnp.where(m <= cur, 1.0, 0.0)
    h = jnp.dot(xb, wg_ref[...], preferred_element_type=jnp.float32)
    agg = jnp.dot(mask, h, preferred_element_type=jnp.float32) + h
    y = agg * (1.0 / (K + 1)) + bg_ref[...]
    y = y * (gamma_ref[...] * BN_SCALE) + beta_ref[...]
    y = y * 0.5 * (1.0 + jax.lax.erf(y * INV_SQRT2))
    out_ref[0] = y + xb


@jax.jit
def kernel(x, Wg, bg, gamma, beta):
    x_nodes = jnp.transpose(x, (0, 2, 3, 1)).reshape(B, N, C)
    out = pl.pallas_call(
        _graph_kernel,
        grid=(B,),
        in_specs=[
            pl.BlockSpec((1, N, C), lambda b: (b, 0, 0)),
            pl.BlockSpec((C, C), lambda b: (0, 0)),
            pl.BlockSpec((1, C), lambda b: (0, 0)),
            pl.BlockSpec((1, C), lambda b: (0, 0)),
            pl.BlockSpec((1, C), lambda b: (0, 0)),
        ],
        out_specs=pl.BlockSpec((1, N, C), lambda b: (b, 0, 0)),
        out_shape=jax.ShapeDtypeStruct((B, N, C), jnp.float32),
    )(x_nodes, Wg, bg.reshape(1, C), gamma.reshape(1, C), beta.reshape(1, C))
    return out.reshape(B, H, W_, C).transpose(0, 3, 1, 2)
